# baseline (speedup 1.0000x reference)
import dataclasses

import jax
import jax.numpy as jnp
from jax import lax
from jax.experimental import pallas as pl
from jax.experimental.pallas import tpu as pltpu
from jax.experimental.pallas import tpu_sc as plsc

_C = 1024
_NW = 32
_CHUNK = 4000
_NCHUNK = 1000
_VECS = _CHUNK // 16
_EPS = 1e-07


def _sc_compiler_params():
    cp = pltpu.CompilerParams()
    if "needs_layout_passes" in pltpu.CompilerParams.__dataclass_fields__:
        cp = dataclasses.replace(cp, needs_layout_passes=False)
    return cp


def _sc_histograms(preds, trues):
    mesh = plsc.VectorSubcoreMesh(core_axis_name="c", subcore_axis_name="s")

    @pl.kernel(
        compiler_params=_sc_compiler_params(),
        out_type=jax.ShapeDtypeStruct((3, _NW, _C), jnp.int32),
        mesh=mesh,
        scratch_types=[
            pltpu.VMEM((16 * 2 * _C,), jnp.int32),
            pltpu.VMEM((16 * _C,), jnp.int32),
            pltpu.VMEM((_C,), jnp.int32),
            pltpu.VMEM((_C,), jnp.int32),
            pltpu.VMEM((_C,), jnp.int32),
        ],
    )
    def hist_kernel(p_hbm, t_hbm, out_hbm, hc_ref, ht_ref, alo_ref, ahi_ref,
                    at_ref):
        zeros16 = jnp.zeros((16,), jnp.int32)

        @plsc.parallel_loop(0, 2 * _C * 16, step=16, unroll=8)
        def _(i):
            hc_ref[pl.ds(i, 16)] = zeros16

        @plsc.parallel_loop(0, _C * 16, step=16, unroll=8)
        def _(i):
            ht_ref[pl.ds(i, 16)] = zeros16

        lane = lax.iota(jnp.int32, 16)
        base_c = lane * (2 * _C)
        base_t = lane * _C
        ones = jnp.ones((16,), jnp.int32)

        def chunk_body(p_vmem, t_vmem):
            @plsc.parallel_loop(0, _CHUNK, step=16, unroll=8)
            def _(v):
                p = p_vmem[pl.ds(v, 16)]
                t = t_vmem[pl.ds(v, 16)]
                match = (p == t).astype(jnp.int32)
                idx_c = base_c + p + match * _C
                idx_t = base_t + t
                plsc.addupdate_scatter(hc_ref, [idx_c], ones)
                plsc.addupdate_scatter(ht_ref, [idx_t], ones)

        pltpu.emit_pipeline(
            chunk_body,
            grid=(_NCHUNK,),
            in_specs=[
                pl.BlockSpec((_CHUNK,), lambda i: (i,)),
                pl.BlockSpec((_CHUNK,), lambda i: (i,)),
            ],
            core_axis_name=("c", "s"),
            dimension_semantics=(pltpu.PARALLEL,),
        )(p_hbm, t_hbm)

        @plsc.parallel_loop(0, _C, step=16)
        def _(c0):
            vlo = jnp.zeros((16,), jnp.int32)
            vhi = jnp.zeros((16,), jnp.int32)
            vt = jnp.zeros((16,), jnp.int32)
            for lane_i in range(16):
                vlo = vlo + hc_ref[pl.ds(lane_i * 2 * _C + c0, 16)]
                vhi = vhi + hc_ref[pl.ds(lane_i * 2 * _C + _C + c0, 16)]
                vt = vt + ht_ref[pl.ds(lane_i * _C + c0, 16)]
            alo_ref[pl.ds(c0, 16)] = vlo
            ahi_ref[pl.ds(c0, 16)] = vhi
            at_ref[pl.ds(c0, 16)] = vt

        wid = lax.axis_index("s") * 2 + lax.axis_index("c")
        pltpu.sync_copy(alo_ref, out_hbm.at[0, wid])
        pltpu.sync_copy(ahi_ref, out_hbm.at[1, wid])
        pltpu.sync_copy(at_ref, out_hbm.at[2, wid])

    return hist_kernel(preds, trues)


def _f1_body(parts_ref, o_ref):
    x = parts_ref[...].astype(jnp.float32)
    s = jnp.sum(x, axis=1)
    lo = s[0:1, :]
    tp = s[1:2, :]
    ht = s[2:3, :]
    hp = lo + tp
    fn = ht - tp
    fp = hp - tp
    precision = tp / (tp + fn + _EPS)
    recall = tp / (tp + fp + _EPS)
    f1 = 2.0 * precision * recall / (precision + recall + _EPS)
    o_ref[...] = jnp.sum(f1, axis=1, keepdims=True) / 1000.0


def _f1_call(parts):
    return pl.pallas_call(
        _f1_body,
        out_shape=jax.ShapeDtypeStruct((1, 1), jnp.float32),
    )(parts)


@jax.jit
def kernel(preds, trues):
    parts = _sc_histograms(preds, trues)
    out = _f1_call(parts)
    return out[0, 0]


# speedup vs baseline: 1.0523x; 1.0523x over previous
import dataclasses

import jax
import jax.numpy as jnp
from jax import lax
from jax.experimental import pallas as pl
from jax.experimental.pallas import tpu as pltpu
from jax.experimental.pallas import tpu_sc as plsc

_C = 1024
_NW = 32
_CHUNK = 8000
_NCHUNK = 500
_VECS = _CHUNK // 16
_EPS = 1e-07


def _sc_compiler_params():
    cp = pltpu.CompilerParams()
    if "needs_layout_passes" in pltpu.CompilerParams.__dataclass_fields__:
        cp = dataclasses.replace(cp, needs_layout_passes=False)
    return cp


def _sc_histograms(preds, trues):
    mesh = plsc.VectorSubcoreMesh(core_axis_name="c", subcore_axis_name="s")

    @pl.kernel(
        compiler_params=_sc_compiler_params(),
        out_type=jax.ShapeDtypeStruct((3, _NW, _C), jnp.int32),
        mesh=mesh,
        scratch_types=[
            pltpu.VMEM((16 * 2 * _C,), jnp.int32),
            pltpu.VMEM((16 * _C,), jnp.int32),
            pltpu.VMEM((_C,), jnp.int32),
            pltpu.VMEM((_C,), jnp.int32),
            pltpu.VMEM((_C,), jnp.int32),
        ],
    )
    def hist_kernel(p_hbm, t_hbm, out_hbm, hc_ref, ht_ref, alo_ref, ahi_ref,
                    at_ref):
        zeros16 = jnp.zeros((16,), jnp.int32)

        @plsc.parallel_loop(0, 2 * _C * 16, step=16, unroll=8)
        def _(i):
            hc_ref[pl.ds(i, 16)] = zeros16

        @plsc.parallel_loop(0, _C * 16, step=16, unroll=8)
        def _(i):
            ht_ref[pl.ds(i, 16)] = zeros16

        lane = lax.iota(jnp.int32, 16)
        base_c = lane * (2 * _C)
        base_t = lane * _C
        ones = jnp.ones((16,), jnp.int32)

        def chunk_body(p_vmem, t_vmem):
            @plsc.parallel_loop(0, _CHUNK, step=16, unroll=8)
            def _(v):
                p = p_vmem[pl.ds(v, 16)]
                t = t_vmem[pl.ds(v, 16)]
                match = (p == t).astype(jnp.int32)
                idx_c = base_c + p + match * _C
                idx_t = base_t + t
                plsc.addupdate_scatter(hc_ref, [idx_c], ones)
                plsc.addupdate_scatter(ht_ref, [idx_t], ones)

        pltpu.emit_pipeline(
            chunk_body,
            grid=(_NCHUNK,),
            in_specs=[
                pl.BlockSpec((_CHUNK,), lambda i: (i,)),
                pl.BlockSpec((_CHUNK,), lambda i: (i,)),
            ],
            core_axis_name=("c", "s"),
            dimension_semantics=(pltpu.PARALLEL,),
        )(p_hbm, t_hbm)

        @plsc.parallel_loop(0, _C, step=16)
        def _(c0):
            vlo = jnp.zeros((16,), jnp.int32)
            vhi = jnp.zeros((16,), jnp.int32)
            vt = jnp.zeros((16,), jnp.int32)
            for lane_i in range(16):
                vlo = vlo + hc_ref[pl.ds(lane_i * 2 * _C + c0, 16)]
                vhi = vhi + hc_ref[pl.ds(lane_i * 2 * _C + _C + c0, 16)]
                vt = vt + ht_ref[pl.ds(lane_i * _C + c0, 16)]
            alo_ref[pl.ds(c0, 16)] = vlo
            ahi_ref[pl.ds(c0, 16)] = vhi
            at_ref[pl.ds(c0, 16)] = vt

        wid = lax.axis_index("s") * 2 + lax.axis_index("c")
        pltpu.sync_copy(alo_ref, out_hbm.at[0, wid])
        pltpu.sync_copy(ahi_ref, out_hbm.at[1, wid])
        pltpu.sync_copy(at_ref, out_hbm.at[2, wid])

    return hist_kernel(preds, trues)


def _f1_body(parts_ref, o_ref):
    x = parts_ref[...].astype(jnp.float32)
    s = jnp.sum(x, axis=1)
    lo = s[0:1, :]
    tp = s[1:2, :]
    ht = s[2:3, :]
    hp = lo + tp
    fn = ht - tp
    fp = hp - tp
    precision = tp / (tp + fn + _EPS)
    recall = tp / (tp + fp + _EPS)
    f1 = 2.0 * precision * recall / (precision + recall + _EPS)
    o_ref[...] = jnp.sum(f1, axis=1, keepdims=True) / 1000.0


def _f1_call(parts):
    return pl.pallas_call(
        _f1_body,
        out_shape=jax.ShapeDtypeStruct((1, 1), jnp.float32),
    )(parts)


@jax.jit
def kernel(preds, trues):
    parts = _sc_histograms(preds, trues)
    out = _f1_call(parts)
    return out[0, 0]
